# Initial kernel scaffold; baseline (speedup 1.0000x reference)
import jax
import jax.numpy as jnp
from jax.experimental import pallas as pl


def kernel(token_ids, embedding_matrix):
    raise NotImplementedError("write your pallas kernel here")


# speedup vs baseline: 1.5089x; 1.5089x over previous
import functools

import jax
import jax.numpy as jnp
from jax import lax
from jax.experimental import pallas as pl
from jax.experimental.pallas import tpu as pltpu
from jax.experimental.pallas import tpu_sc as plsc

D = 32
NW = 32
CHUNK = 2048


def _make_gather(b_total: int):
    b_per_w = b_total // NW
    n_chunks = b_per_w // CHUNK
    mesh = plsc.VectorSubcoreMesh(core_axis_name="c", subcore_axis_name="s")

    @functools.partial(
        pl.kernel,
        mesh=mesh,
        compiler_params=pltpu.CompilerParams(use_tc_tiling_on_sc=False),
        out_type=jax.ShapeDtypeStruct((b_total, D), jnp.float32),
        scratch_types=[
            pltpu.VMEM((b_per_w,), jnp.int32),
            pltpu.VMEM((CHUNK, D), jnp.float32),
            pltpu.SemaphoreType.DMA,
        ],
    )
    def gather(idx_hbm, table_hbm, out_hbm, idx_v, rows_v, sem):
        wid = lax.axis_index("s") * 2 + lax.axis_index("c")
        base = wid * b_per_w
        pltpu.sync_copy(idx_hbm.at[pl.ds(base, b_per_w)], idx_v)
        for i in range(n_chunks):
            pltpu.async_copy(
                table_hbm.at[idx_v.at[pl.ds(i * CHUNK, CHUNK)]], rows_v, sem
            ).wait()
            pltpu.sync_copy(rows_v, out_hbm.at[pl.ds(base + i * CHUNK, CHUNK)])

    return gather


def kernel(token_ids, embedding_matrix):
    b, s = token_ids.shape
    flat = token_ids.reshape(b * s).astype(jnp.int32)
    out = _make_gather(b * s)(flat, embedding_matrix)
    return out.reshape(b, s, D)


# speedup vs baseline: 1.5211x; 1.0081x over previous
import functools

import jax
import jax.numpy as jnp
from jax import lax
from jax.experimental import pallas as pl
from jax.experimental.pallas import tpu as pltpu
from jax.experimental.pallas import tpu_sc as plsc

D = 32
NW = 32
CHUNK = 1024
NBUF = 3


def _make_gather(b_total: int):
    b_per_w = b_total // NW
    n_chunks = b_per_w // CHUNK
    mesh = plsc.VectorSubcoreMesh(core_axis_name="c", subcore_axis_name="s")

    @functools.partial(
        pl.kernel,
        mesh=mesh,
        compiler_params=pltpu.CompilerParams(use_tc_tiling_on_sc=False),
        out_type=jax.ShapeDtypeStruct((b_total, D), jnp.float32),
        scratch_types=[
            pltpu.VMEM((b_per_w,), jnp.int32),
            pltpu.VMEM((NBUF, CHUNK, D), jnp.float32),
            pltpu.SemaphoreType.DMA((NBUF,)),
            pltpu.SemaphoreType.DMA((NBUF,)),
        ],
    )
    def gather(idx_hbm, table_hbm, out_hbm, idx_v, rows_v, gsem, ssem):
        wid = lax.axis_index("s") * 2 + lax.axis_index("c")
        base = wid * b_per_w
        pltpu.sync_copy(idx_hbm.at[pl.ds(base, b_per_w)], idx_v)

        def start_gather(i, b):
            return pltpu.async_copy(
                table_hbm.at[idx_v.at[pl.ds(i * CHUNK, CHUNK)]],
                rows_v.at[b],
                gsem.at[b],
            )

        gcp = [None] * n_chunks
        scp = [None] * n_chunks
        for i in range(min(NBUF, n_chunks)):
            gcp[i] = start_gather(i, i)
        for i in range(n_chunks):
            b = i % NBUF
            gcp[i].wait()
            scp[i] = pltpu.async_copy(
                rows_v.at[b],
                out_hbm.at[pl.ds(base + i * CHUNK, CHUNK)],
                ssem.at[b],
            )
            nxt = i + NBUF
            if nxt < n_chunks:
                scp[i].wait()
                gcp[nxt] = start_gather(nxt, b)
        for i in range(max(0, n_chunks - NBUF), n_chunks):
            scp[i].wait()

    return gather


def kernel(token_ids, embedding_matrix):
    b, s = token_ids.shape
    flat = token_ids.reshape(b * s).astype(jnp.int32)
    out = _make_gather(b * s)(flat, embedding_matrix)
    return out.reshape(b, s, D)
